# Initial kernel scaffold; baseline (speedup 1.0000x reference)
import jax
import jax.numpy as jnp
from jax.experimental import pallas as pl


def kernel(u, pos_v, neg_v, center_w, context_w):
    raise NotImplementedError("write your pallas kernel here")


# speedup vs baseline: 4.8813x; 4.8813x over previous
import jax
import jax.numpy as jnp
from jax import lax
from jax.experimental import pallas as pl
from jax.experimental.pallas import tpu as pltpu
from jax.experimental.pallas import tpu_sc as plsc
import functools

NUM_CORES = 2
NUM_SUBCORES = 16
NW = NUM_CORES * NUM_SUBCORES
LANES = 16

D = 64
CHUNK = 32


def _sc_scores(u_i, pos_i, neg_i, center_w, context_w):
    B = u_i.shape[0]
    K = neg_i.shape[0] // B
    S = K + 1
    bpw = B // NW
    nchunk = bpw // CHUNK
    ck = CHUNK * K
    ngath = ck // 128
    assert B % NW == 0 and bpw % CHUNK == 0 and ck % 128 == 0

    mesh = plsc.VectorSubcoreMesh(
        core_axis_name="c", subcore_axis_name="s",
        num_cores=NUM_CORES, num_subcores=NUM_SUBCORES)

    @functools.partial(
        pl.kernel,
        out_type=jax.ShapeDtypeStruct((B * S,), jnp.float32),
        mesh=mesh,
        scratch_types=[
            pltpu.VMEM((bpw,), jnp.int32),
            pltpu.VMEM((bpw,), jnp.int32),
            pltpu.VMEM((bpw * K,), jnp.int32),
            pltpu.VMEM((CHUNK, D), jnp.float32),
            pltpu.VMEM((CHUNK, D), jnp.float32),
            pltpu.VMEM((CHUNK, D), jnp.float32),
            pltpu.VMEM((CHUNK, D), jnp.float32),
            pltpu.VMEM((ck, D), jnp.float32),
            pltpu.VMEM((ck, D), jnp.float32),
            pltpu.VMEM((CHUNK * S,), jnp.float32),
            pltpu.VMEM((CHUNK * S,), jnp.float32),
            pltpu.SemaphoreType.DMA,
            pltpu.SemaphoreType.DMA,
        ],
        compiler_params=pltpu.CompilerParams(
            needs_layout_passes=False, use_tc_tiling_on_sc=False),
    )
    def k(center_hbm, context_hbm, u_hbm, pos_hbm, neg_hbm, out_hbm,
          u_idx, pos_idx, neg_idx,
          ur0, ur1, pr0, pr1, nr0, nr1, sv0, sv1, sem0, sem1):
        wid = lax.axis_index("s") * NUM_CORES + lax.axis_index("c")
        base = wid * bpw
        pltpu.sync_copy(u_hbm.at[pl.ds(base, bpw)], u_idx)
        pltpu.sync_copy(pos_hbm.at[pl.ds(base, bpw)], pos_idx)
        pltpu.sync_copy(neg_hbm.at[pl.ds(base * K, bpw * K)], neg_idx)

        bufs = [(ur0, pr0, nr0, sv0, sem0), (ur1, pr1, nr1, sv1, sem1)]

        def issue(g, ur, pr, nr, sem):
            off = g * CHUNK
            cps = [
                pltpu.async_copy(center_hbm.at[u_idx.at[pl.ds(off, CHUNK)]],
                                 ur, sem),
                pltpu.async_copy(context_hbm.at[pos_idx.at[pl.ds(off, CHUNK)]],
                                 pr, sem),
            ]
            for j in range(ngath):
                cps.append(pltpu.async_copy(
                    context_hbm.at[neg_idx.at[pl.ds(g * ck + j * 128, 128)]],
                    nr.at[pl.ds(j * 128, 128)], sem))
            return cps

        lane0 = lax.iota(jnp.int32, LANES) == 0

        def compute(g, ur, pr, nr, sv):
            def emit(sv, s, pos):
                idxv = jnp.broadcast_to(jnp.int32(0) + pos, (LANES,))
                val = jnp.broadcast_to(s, (LANES,))
                plsc.store_scatter(sv, [idxv], val, mask=lane0)

            def body(b, carry):
                u0 = ur[b, pl.ds(0, LANES)]
                u1 = ur[b, pl.ds(LANES, LANES)]
                u2 = ur[b, pl.ds(2 * LANES, LANES)]
                u3 = ur[b, pl.ds(3 * LANES, LANES)]

                def dot(rref, row):
                    acc = (rref[row, pl.ds(0, LANES)] * u0
                           + rref[row, pl.ds(LANES, LANES)] * u1
                           + rref[row, pl.ds(2 * LANES, LANES)] * u2
                           + rref[row, pl.ds(3 * LANES, LANES)] * u3)
                    return jnp.sum(acc)

                sbase = b * S
                emit(sv, dot(pr, b), sbase)
                for kk in range(K):
                    emit(sv, dot(nr, b * K + kk), sbase + 1 + kk)
                return carry

            lax.fori_loop(0, CHUNK, body, 0)
            pltpu.sync_copy(
                sv, out_hbm.at[pl.ds(base * S + g * CHUNK * S, CHUNK * S)])

        pend = [None, None]
        pend[0] = issue(0, *bufs[0][:3], bufs[0][4])
        for g in range(nchunk):
            p = g % 2
            if g + 1 < nchunk:
                np_ = (g + 1) % 2
                pend[np_] = issue(g + 1, *bufs[np_][:3], bufs[np_][4])
            for cp in pend[p]:
                cp.wait()
            compute(g, bufs[p][0], bufs[p][1], bufs[p][2], bufs[p][3])

    return k(center_w, context_w, u_i, pos_i, neg_i)


def _tc_loss(scores2d, B, S):
    def body(s_ref, o_ref):
        x = s_ref[...]
        ridx = lax.broadcasted_iota(jnp.int32, x.shape, 0)
        cidx = lax.broadcasted_iota(jnp.int32, x.shape, 1)
        flat = ridx * x.shape[1] + cidx
        is_pos = (flat % S) == 0
        xs = jnp.where(is_pos, x, -x)
        sig = 1.0 / (1.0 + jnp.exp(-xs))
        t = -jnp.log(sig + 1e-8)
        o_ref[...] = jnp.broadcast_to(jnp.sum(t) / B, (1, 1))

    return pl.pallas_call(
        body,
        out_shape=jax.ShapeDtypeStruct((1, 1), jnp.float32),
    )(scores2d)


def kernel(u, pos_v, neg_v, center_w, context_w):
    B = u.shape[0]
    K = neg_v.shape[1]
    S = K + 1
    u_i = u.astype(jnp.int32)
    pos_i = pos_v.astype(jnp.int32)
    neg_i = neg_v.reshape(-1).astype(jnp.int32)
    scores = _sc_scores(u_i, pos_i, neg_i, center_w, context_w)
    loss = _tc_loss(scores.reshape(B * S // 128, 128), B, S)
    return loss[0, 0]


# speedup vs baseline: 6.7601x; 1.3849x over previous
import jax
import jax.numpy as jnp
from jax import lax
from jax.experimental import pallas as pl
from jax.experimental.pallas import tpu as pltpu
from jax.experimental.pallas import tpu_sc as plsc
import functools

NUM_CORES = 2
NUM_SUBCORES = 16
NW = NUM_CORES * NUM_SUBCORES
LANES = 16

D = 64
CHUNK = 16


def _sc_scores(u_i, pos_i, neg_i, cw, xw):
    B = u_i.shape[0]
    K = neg_i.shape[0] // B
    S = K + 1
    bpw = B // NW
    nchunk = bpw // CHUNK
    ck = CHUNK * K
    assert B % NW == 0 and bpw % CHUNK == 0 and nchunk % 2 == 0
    assert ck % LANES == 0

    mesh = plsc.VectorSubcoreMesh(
        core_axis_name="c", subcore_axis_name="s",
        num_cores=NUM_CORES, num_subcores=NUM_SUBCORES)

    @functools.partial(
        pl.kernel,
        out_type=jax.ShapeDtypeStruct((B * S,), jnp.float32),
        mesh=mesh,
        scratch_types=[
            pltpu.VMEM((bpw,), jnp.int32),
            pltpu.VMEM((bpw,), jnp.int32),
            pltpu.VMEM((bpw * K,), jnp.int32),
            pltpu.VMEM((CHUNK, D), jnp.float32),
            pltpu.VMEM((CHUNK, D), jnp.float32),
            pltpu.VMEM((CHUNK, D), jnp.float32),
            pltpu.VMEM((CHUNK, D), jnp.float32),
            pltpu.VMEM((ck, D), jnp.float32),
            pltpu.VMEM((ck, D), jnp.float32),
            pltpu.VMEM((CHUNK * S,), jnp.float32),
            pltpu.VMEM((CHUNK * S,), jnp.float32),
            pltpu.SemaphoreType.DMA,
            pltpu.SemaphoreType.DMA,
        ],
        compiler_params=pltpu.CompilerParams(
            needs_layout_passes=False, use_tc_tiling_on_sc=True),
    )
    def k(cw_hbm, xw_hbm, u_hbm, pg_hbm, ng_hbm,
          out_hbm,
          u_idx, pos_idx, neg_idx,
          ur0, ur1, pr0, pr1, nr0, nr1, sv0, sv1, sem0, sem1):
        wid = lax.axis_index("s") * NUM_CORES + lax.axis_index("c")
        base = wid * bpw
        pltpu.sync_copy(u_hbm.at[pl.ds(base, bpw)], u_idx)
        pltpu.sync_copy(pg_hbm.at[pl.ds(base, bpw)], pos_idx)
        pltpu.sync_copy(ng_hbm.at[pl.ds(base * K, bpw * K)], neg_idx)

        bufs = [(ur0, pr0, nr0, sv0, sem0), (ur1, pr1, nr1, sv1, sem1)]

        def issue(g, ur, pr, nr, sem):
            off = pl.multiple_of(g * CHUNK, CHUNK)
            iv = u_idx[pl.ds(off, CHUNK)]
            pv = pos_idx[pl.ds(off, CHUNK)]
            for b in range(CHUNK):
                pltpu.async_copy(cw_hbm.at[iv[b]], ur.at[b], sem)
                pltpu.async_copy(xw_hbm.at[pv[b]], pr.at[b], sem)
            noff = pl.multiple_of(g * ck, 8)

            def ngrp(kg, carry):
                nbase = pl.multiple_of(kg * LANES, LANES)
                nv = neg_idx[pl.ds(noff + nbase, LANES)]
                for b in range(LANES):
                    pltpu.async_copy(xw_hbm.at[nv[b]], nr.at[nbase + b], sem)
                return carry

            lax.fori_loop(0, ck // LANES, ngrp, 0)

        def drain(ur, pr, nr, sem):
            pltpu.make_async_copy(cw_hbm.at[pl.ds(0, CHUNK)], ur, sem).wait()
            pltpu.make_async_copy(cw_hbm.at[pl.ds(0, CHUNK)], pr, sem).wait()
            pltpu.make_async_copy(cw_hbm.at[pl.ds(0, ck)], nr, sem).wait()

        lane0 = lax.iota(jnp.int32, LANES) == 0

        def compute(g, ur, pr, nr, sv):
            def emit(sv, s, pos):
                idxv = jnp.broadcast_to(jnp.int32(0) + pos, (LANES,))
                val = jnp.broadcast_to(s, (LANES,))
                plsc.store_scatter(sv, [idxv], val, mask=lane0)

            def body(b, carry):
                u0 = ur[b, pl.ds(0, LANES)]
                u1 = ur[b, pl.ds(LANES, LANES)]
                u2 = ur[b, pl.ds(2 * LANES, LANES)]
                u3 = ur[b, pl.ds(3 * LANES, LANES)]

                def dot(rref, row):
                    acc = (rref[row, pl.ds(0, LANES)] * u0
                           + rref[row, pl.ds(LANES, LANES)] * u1
                           + rref[row, pl.ds(2 * LANES, LANES)] * u2
                           + rref[row, pl.ds(3 * LANES, LANES)] * u3)
                    return jnp.sum(acc)

                sbase = b * S
                emit(sv, dot(pr, b), sbase)
                for kk in range(K):
                    emit(sv, dot(nr, b * K + kk), sbase + 1 + kk)
                return carry

            lax.fori_loop(0, CHUNK, body, 0)
            soff = pl.multiple_of(g * (CHUNK * S), 8)
            pltpu.sync_copy(sv, out_hbm.at[pl.ds(base * S + soff, CHUNK * S)])

        def step(g, p):
            ur, pr, nr, sv, sem = bufs[p]
            drain(ur, pr, nr, sem)
            compute(g, ur, pr, nr, sv)

        issue(0, *bufs[0][:3], bufs[0][4])

        def pair(i, carry):
            g = 2 * i
            issue(g + 1, *bufs[1][:3], bufs[1][4])
            step(g, 0)
            issue(g + 2, *bufs[0][:3], bufs[0][4])
            step(g + 1, 1)
            return carry

        lax.fori_loop(0, nchunk // 2 - 1, pair, 0)
        g_last = nchunk - 2
        issue(g_last + 1, *bufs[1][:3], bufs[1][4])
        step(g_last, 0)
        step(g_last + 1, 1)

    return k(cw, xw, u_i, pos_i, neg_i)


def _tc_loss(scores2d, B, S):
    def body(s_ref, o_ref):
        x = s_ref[...]
        ridx = lax.broadcasted_iota(jnp.int32, x.shape, 0)
        cidx = lax.broadcasted_iota(jnp.int32, x.shape, 1)
        flat = ridx * x.shape[1] + cidx
        is_pos = (flat % S) == 0
        xs = jnp.where(is_pos, x, -x)
        sig = 1.0 / (1.0 + jnp.exp(-xs))
        t = -jnp.log(sig + 1e-8)
        o_ref[...] = jnp.broadcast_to(jnp.sum(t) / B, (1, 1))

    return pl.pallas_call(
        body,
        out_shape=jax.ShapeDtypeStruct((1, 1), jnp.float32),
    )(scores2d)


def kernel(u, pos_v, neg_v, center_w, context_w):
    B = u.shape[0]
    K = neg_v.shape[1]
    S = K + 1
    u_i = u.astype(jnp.int32)
    pos_i = pos_v.astype(jnp.int32)
    neg_i = neg_v.reshape(-1).astype(jnp.int32)
    scores = _sc_scores(u_i, pos_i, neg_i, center_w, context_w)
    loss = _tc_loss(scores.reshape(B * S // 128, 128), B, S)
    return loss[0, 0]
